# initial kernel scaffold (unmeasured)
import jax
import jax.numpy as jnp
from jax import lax
from jax.experimental import pallas as pl
from jax.experimental.pallas import tpu as pltpu


def kernel(
    x,
):
    def body(*refs):
        pass

    out_shape = jax.ShapeDtypeStruct(..., jnp.float32)
    return pl.pallas_call(body, out_shape=out_shape)(...)


# baseline (device time: 196527 ns/iter reference)
import jax
import jax.numpy as jnp
from jax import lax
from jax.experimental import pallas as pl
from jax.experimental.pallas import tpu as pltpu


def kernel(x):
    m, n = x.shape

    def body(x_ref, out_ref, comm_ref, send_sem, recv_sem):
        my_x = lax.axis_index("x")
        my_y = lax.axis_index("y")
        peer = (1 - my_x, my_y)

        barrier_sem = pltpu.get_barrier_semaphore()
        pl.semaphore_signal(
            barrier_sem, inc=1,
            device_id=peer, device_id_type=pl.DeviceIdType.MESH,
        )
        pl.semaphore_wait(barrier_sem, 1)

        rdma = pltpu.make_async_remote_copy(
            src_ref=x_ref,
            dst_ref=comm_ref,
            send_sem=send_sem,
            recv_sem=recv_sem,
            device_id=peer,
            device_id_type=pl.DeviceIdType.MESH,
        )
        rdma.start()
        rdma.wait()

        out_ref[...] = x_ref[...] + comm_ref[...]

    return pl.pallas_call(
        body,
        out_shape=jax.ShapeDtypeStruct((m, n), x.dtype),
        in_specs=[pl.BlockSpec(memory_space=pltpu.VMEM)],
        out_specs=pl.BlockSpec(memory_space=pltpu.VMEM),
        scratch_shapes=[
            pltpu.VMEM((m, n), x.dtype),
            pltpu.SemaphoreType.DMA,
            pltpu.SemaphoreType.DMA,
        ],
        compiler_params=pltpu.CompilerParams(collective_id=0),
    )(x)


# device time: 113662 ns/iter; 1.7290x vs baseline; 1.7290x over previous
import jax
import jax.numpy as jnp
from jax import lax
from jax.experimental import pallas as pl
from jax.experimental.pallas import tpu as pltpu

C = 16


def kernel(x):
    m, n = x.shape
    half = m // 2
    ck = half // C

    def body(x_ref, out_ref, comm_ref, s1, r1, s2, r2):
        my_x = lax.axis_index("x")
        my_y = lax.axis_index("y")
        x_peer = (1 - my_x, my_y)
        y_peer = (my_x, 1 - my_y)

        barrier_sem = pltpu.get_barrier_semaphore()
        for nbr in (x_peer, y_peer):
            pl.semaphore_signal(
                barrier_sem, inc=1,
                device_id=nbr, device_id_type=pl.DeviceIdType.MESH,
            )
        pl.semaphore_wait(barrier_sem, 2)

        mine0 = my_y * half
        other0 = (1 - my_y) * half

        p1 = []
        for c in range(C):
            rdma = pltpu.make_async_remote_copy(
                src_ref=x_ref.at[pl.ds(mine0 + c * ck, ck), :],
                dst_ref=comm_ref.at[c],
                send_sem=s1.at[c],
                recv_sem=r1.at[c],
                device_id=x_peer,
                device_id_type=pl.DeviceIdType.MESH,
            )
            rdma.start()
            p1.append(rdma)

        p2 = []
        for c in range(C):
            p1[c].wait_recv()
            rows = pl.ds(mine0 + c * ck, ck)
            out_ref[rows, :] = x_ref[rows, :] + comm_ref[c, :, :]
            rdma = pltpu.make_async_remote_copy(
                src_ref=out_ref.at[rows, :],
                dst_ref=out_ref.at[rows, :],
                send_sem=s2.at[c],
                recv_sem=r2.at[c],
                device_id=y_peer,
                device_id_type=pl.DeviceIdType.MESH,
            )
            rdma.start()
            p2.append(rdma)

        for c in range(C):
            rows = pl.ds(other0 + c * ck, ck)
            recv = pltpu.make_async_remote_copy(
                src_ref=out_ref.at[rows, :],
                dst_ref=out_ref.at[rows, :],
                send_sem=s2.at[c],
                recv_sem=r2.at[c],
                device_id=y_peer,
                device_id_type=pl.DeviceIdType.MESH,
            )
            recv.wait_recv()

        for c in range(C):
            p1[c].wait_send()
            p2[c].wait_send()

    return pl.pallas_call(
        body,
        out_shape=jax.ShapeDtypeStruct((m, n), x.dtype),
        in_specs=[pl.BlockSpec(memory_space=pltpu.VMEM)],
        out_specs=pl.BlockSpec(memory_space=pltpu.VMEM),
        scratch_shapes=[
            pltpu.VMEM((C, ck, n), x.dtype),
            pltpu.SemaphoreType.DMA((C,)),
            pltpu.SemaphoreType.DMA((C,)),
            pltpu.SemaphoreType.DMA((C,)),
            pltpu.SemaphoreType.DMA((C,)),
        ],
        compiler_params=pltpu.CompilerParams(collective_id=0),
    )(x)


# device time: 107035 ns/iter; 1.8361x vs baseline; 1.0619x over previous
import jax
import jax.numpy as jnp
from jax import lax
from jax.experimental import pallas as pl
from jax.experimental.pallas import tpu as pltpu

C = 32


def kernel(x):
    m, n = x.shape
    half = m // 2
    ck = half // C

    def body(x_ref, out_ref, xh_ref, comm_ref, red_ref,
             li, lo, s1, r1, s2, r2):
        my_x = lax.axis_index("x")
        my_y = lax.axis_index("y")
        x_peer = (1 - my_x, my_y)
        y_peer = (my_x, 1 - my_y)

        barrier_sem = pltpu.get_barrier_semaphore()
        for nbr in (x_peer, y_peer):
            pl.semaphore_signal(
                barrier_sem, inc=1,
                device_id=nbr, device_id_type=pl.DeviceIdType.MESH,
            )
        pl.semaphore_wait(barrier_sem, 2)

        mine0 = my_y * half
        other0 = (1 - my_y) * half

        p1 = []
        lcp = []
        for c in range(C):
            rows = pl.ds(mine0 + c * ck, ck)
            rdma = pltpu.make_async_remote_copy(
                src_ref=x_ref.at[rows, :],
                dst_ref=comm_ref.at[c],
                send_sem=s1.at[c],
                recv_sem=r1.at[c],
                device_id=x_peer,
                device_id_type=pl.DeviceIdType.MESH,
            )
            rdma.start()
            p1.append(rdma)
            cp = pltpu.make_async_copy(x_ref.at[rows, :], xh_ref.at[c], li.at[c])
            cp.start()
            lcp.append(cp)

        p2 = []
        ocp = []
        for c in range(C):
            p1[c].wait_recv()
            lcp[c].wait()
            red_ref[c, :, :] = xh_ref[c, :, :] + comm_ref[c, :, :]
            rows = pl.ds(mine0 + c * ck, ck)
            rdma = pltpu.make_async_remote_copy(
                src_ref=red_ref.at[c],
                dst_ref=out_ref.at[rows, :],
                send_sem=s2.at[c],
                recv_sem=r2.at[c],
                device_id=y_peer,
                device_id_type=pl.DeviceIdType.MESH,
            )
            rdma.start()
            p2.append(rdma)
            cp = pltpu.make_async_copy(red_ref.at[c], out_ref.at[rows, :], lo.at[c])
            cp.start()
            ocp.append(cp)

        for c in range(C):
            rows = pl.ds(other0 + c * ck, ck)
            recv = pltpu.make_async_remote_copy(
                src_ref=red_ref.at[c],
                dst_ref=out_ref.at[rows, :],
                send_sem=s2.at[c],
                recv_sem=r2.at[c],
                device_id=y_peer,
                device_id_type=pl.DeviceIdType.MESH,
            )
            recv.wait_recv()

        for c in range(C):
            ocp[c].wait()
            p1[c].wait_send()
            p2[c].wait_send()

    return pl.pallas_call(
        body,
        out_shape=jax.ShapeDtypeStruct((m, n), x.dtype),
        in_specs=[pl.BlockSpec(memory_space=pl.ANY)],
        out_specs=pl.BlockSpec(memory_space=pl.ANY),
        scratch_shapes=[
            pltpu.VMEM((C, ck, n), x.dtype),
            pltpu.VMEM((C, ck, n), x.dtype),
            pltpu.VMEM((C, ck, n), x.dtype),
            pltpu.SemaphoreType.DMA((C,)),
            pltpu.SemaphoreType.DMA((C,)),
            pltpu.SemaphoreType.DMA((C,)),
            pltpu.SemaphoreType.DMA((C,)),
            pltpu.SemaphoreType.DMA((C,)),
            pltpu.SemaphoreType.DMA((C,)),
        ],
        compiler_params=pltpu.CompilerParams(collective_id=0),
    )(x)
